# Initial kernel scaffold; baseline (speedup 1.0000x reference)
import jax
import jax.numpy as jnp
from jax.experimental import pallas as pl


def kernel(x, edge_index, edge_attr, params):
    raise NotImplementedError("write your pallas kernel here")


# speedup vs baseline: 1.5783x; 1.5783x over previous
import functools

import jax
import jax.numpy as jnp
from jax import lax
from jax.experimental import pallas as pl
from jax.experimental.pallas import tpu as pltpu
from jax.experimental.pallas import tpu_sc as plsc

N = 10000
H = 128
NC = 2
NS = 16
NW = NC * NS
CHUNK = 128
CPW = 80
E_PAD = NW * CPW * CHUNK
NACC = 10240
RE = 2048
RN = 2000

_f32 = jnp.float32


def _mesh():
    return plsc.VectorSubcoreMesh(
        core_axis_name="c", subcore_axis_name="s", num_cores=NC, num_subcores=NS
    )


@functools.partial(
    pl.kernel,
    out_type=(
        jax.ShapeDtypeStruct((E_PAD, H), _f32),
        jax.ShapeDtypeStruct((E_PAD, H), _f32),
    ),
    mesh=_mesh(),
    scratch_types=[
        pltpu.VMEM((CPW, CHUNK), jnp.int32),
        pltpu.VMEM((CPW, CHUNK), jnp.int32),
        pltpu.VMEM((CHUNK, H), _f32),
        pltpu.VMEM((CHUNK, H), _f32),
        pltpu.SemaphoreType.DMA,
        pltpu.SemaphoreType.DMA,
    ],
)
def _sc_gather(a_hbm, b_hbm, si_hbm, di_hbm, ga_hbm, gb_hbm,
               sidx, didx, bufa, bufb, sema, semb):
    c = lax.axis_index("c")
    s = lax.axis_index("s")
    wid = s * NC + c
    base = wid * CPW
    pltpu.sync_copy(si_hbm.at[pl.ds(base, CPW)], sidx)
    pltpu.sync_copy(di_hbm.at[pl.ds(base, CPW)], didx)

    def step(j, carry):
        ca = pltpu.async_copy(a_hbm.at[sidx.at[j]], bufa, sema)
        cb = pltpu.async_copy(b_hbm.at[didx.at[j]], bufb, semb)
        ca.wait()
        cb.wait()
        row0 = (base + j) * CHUNK
        pltpu.sync_copy(bufa, ga_hbm.at[pl.ds(row0, CHUNK)])
        pltpu.sync_copy(bufb, gb_hbm.at[pl.ds(row0, CHUNK)])
        return carry

    lax.fori_loop(0, CPW, step, 0)


@functools.partial(
    pl.kernel,
    out_type=jax.ShapeDtypeStruct((NC, NACC, H), _f32),
    mesh=_mesh(),
    scratch_types=[
        pltpu.VMEM((CPW, CHUNK), jnp.int32),
        pltpu.VMEM((CHUNK, H), _f32),
        pltpu.VMEM_SHARED((NACC, H), _f32),
    ],
)
def _sc_scatter(e_hbm, di_hbm, out_hbm, didx, rows, acc):
    c = lax.axis_index("c")
    s = lax.axis_index("s")
    wid = s * NC + c
    base = wid * CPW

    def zrow(i, carry):
        def zcol(k, carry2):
            rows[i, pl.ds(k * 16, 16)] = jnp.zeros((16,), _f32)
            return carry2
        return lax.fori_loop(0, H // 16, zcol, carry)

    lax.fori_loop(0, CHUNK, zrow, 0)
    rpw = NACC // NS
    for t in range(rpw // CHUNK):
        pltpu.sync_copy(rows, acc.at[pl.ds(s * rpw + t * CHUNK, CHUNK)])
    plsc.subcore_barrier()

    pltpu.sync_copy(di_hbm.at[pl.ds(base, CPW)], didx)

    def step(j, carry):
        row0 = (base + j) * CHUNK
        pltpu.sync_copy(e_hbm.at[pl.ds(row0, CHUNK)], rows)
        pltpu.sync_copy(rows, acc.at[didx.at[j]], add=True)
        return carry

    lax.fori_loop(0, CPW, step, 0)
    plsc.subcore_barrier()
    pltpu.sync_copy(acc.at[pl.ds(s * rpw, rpw)], out_hbm.at[c, pl.ds(s * rpw, rpw)])


def _dot(x, w):
    return jnp.dot(x, w, preferred_element_type=_f32,
                   precision=lax.Precision.HIGHEST)


def _ln(h, g, b):
    mu = jnp.mean(h, axis=-1, keepdims=True)
    var = jnp.mean((h - mu) * (h - mu), axis=-1, keepdims=True)
    return (h - mu) * lax.rsqrt(var + 1e-5) * g + b


def _wspec(shape):
    return pl.BlockSpec(shape, lambda i: (0,) * len(shape))


def _rspec(rows, cols):
    return pl.BlockSpec((rows, cols), lambda i: (i, 0))


def _tc_edge_block(ga, gb, e, w1e, b1, w2, b2, w3, b3, g, bln):
    def body(ga_r, gb_r, e_r, w1e_r, b1_r, w2_r, b2_r, w3_r, b3_r, g_r, bln_r, o_r):
        ev = e_r[...]
        h = jnp.maximum(ga_r[...] + gb_r[...] + _dot(ev, w1e_r[...]) + b1_r[...], 0.0)
        h = jnp.maximum(_dot(h, w2_r[...]) + b2_r[...], 0.0)
        h = _dot(h, w3_r[...]) + b3_r[...]
        o_r[...] = ev + _ln(h, g_r[...], bln_r[...])

    return pl.pallas_call(
        body,
        grid=(E_PAD // RE,),
        in_specs=[
            _rspec(RE, H), _rspec(RE, H), _rspec(RE, H),
            _wspec((H, H)), _wspec((1, H)), _wspec((H, H)), _wspec((1, H)),
            _wspec((H, H)), _wspec((1, H)), _wspec((1, H)), _wspec((1, H)),
        ],
        out_specs=_rspec(RE, H),
        out_shape=jax.ShapeDtypeStruct((E_PAD, H), _f32),
    )(ga, gb, e, w1e, b1, w2, b2, w3, b3, g, bln)


def _tc_node_block(node, sk, sp, w1n, w1a, b1, w2, b2, w3, b3, g, bln, wsn, wdn):
    have_next = wsn is not None

    def body(n_r, sk_r, sp_r, w1n_r, w1a_r, b1_r, w2_r, b2_r, w3_r, b3_r,
             g_r, bln_r, *rest):
        if have_next:
            wsn_r, wdn_r, o_r, a_r, b_r = rest
        else:
            (o_r,) = rest
        nv = n_r[...]
        agg = (sk_r[0] + sk_r[1]) - (sp_r[0] + sp_r[1])
        h = jnp.maximum(_dot(nv, w1n_r[...]) + _dot(agg, w1a_r[...]) + b1_r[...], 0.0)
        h = jnp.maximum(_dot(h, w2_r[...]) + b2_r[...], 0.0)
        h = _dot(h, w3_r[...]) + b3_r[...]
        nn = nv + _ln(h, g_r[...], bln_r[...])
        o_r[...] = nn
        if have_next:
            a_r[...] = _dot(nn, wsn_r[...])
            b_r[...] = _dot(nn, wdn_r[...])

    sspec = pl.BlockSpec((NC, RN, H), lambda i: (0, i, 0))
    in_specs = [
        _rspec(RN, H), sspec, sspec,
        _wspec((H, H)), _wspec((H, H)), _wspec((1, H)), _wspec((H, H)),
        _wspec((1, H)), _wspec((H, H)), _wspec((1, H)), _wspec((1, H)),
        _wspec((1, H)),
    ]
    args = [node, sk, sp, w1n, w1a, b1, w2, b2, w3, b3, g, bln]
    nshape = jax.ShapeDtypeStruct((N, H), _f32)
    if have_next:
        in_specs += [_wspec((H, H)), _wspec((H, H))]
        args += [wsn, wdn]
        out_specs = (_rspec(RN, H),) * 3
        out_shape = (nshape, nshape, nshape)
    else:
        out_specs = _rspec(RN, H)
        out_shape = nshape
    return pl.pallas_call(
        body,
        grid=(N // RN,),
        in_specs=in_specs,
        out_specs=out_specs,
        out_shape=out_shape,
    )(*args)


def _tc_enc_node(x, ws, bs, g, bln, wsn, wdn):
    def body(x_r, w0, b0, w1, b1, w2, b2, g_r, bln_r, wsn_r, wdn_r,
             o_r, a_r, b_r):
        h = jnp.maximum(_dot(x_r[...], w0[...]) + b0[...], 0.0)
        h = jnp.maximum(_dot(h, w1[...]) + b1[...], 0.0)
        h = _dot(h, w2[...]) + b2[...]
        nn = _ln(h, g_r[...], bln_r[...])
        o_r[...] = nn
        a_r[...] = _dot(nn, wsn_r[...])
        b_r[...] = _dot(nn, wdn_r[...])

    nshape = jax.ShapeDtypeStruct((N, H), _f32)
    return pl.pallas_call(
        body,
        grid=(N // RN,),
        in_specs=[_rspec(RN, H)] + [
            _wspec((H, H)), _wspec((1, H)), _wspec((H, H)), _wspec((1, H)),
            _wspec((H, H)), _wspec((1, H)), _wspec((1, H)), _wspec((1, H)),
            _wspec((H, H)), _wspec((H, H)),
        ],
        out_specs=(_rspec(RN, H),) * 3,
        out_shape=(nshape, nshape, nshape),
    )(x, ws[0][0], bs[0], ws[1][0], bs[1], ws[2][0], bs[2], g, bln, wsn, wdn)


def _tc_enc_edge(ea, ws, bs, g, bln):
    ein = ea.shape[1]

    def body(e_r, w0, b0, w1, b1, w2, b2, g_r, bln_r, o_r):
        h = jnp.maximum(_dot(e_r[...], w0[...]) + b0[...], 0.0)
        h = jnp.maximum(_dot(h, w1[...]) + b1[...], 0.0)
        h = _dot(h, w2[...]) + b2[...]
        o_r[...] = _ln(h, g_r[...], bln_r[...])

    return pl.pallas_call(
        body,
        grid=(E_PAD // RE,),
        in_specs=[_rspec(RE, ein)] + [
            _wspec((ein, H)), _wspec((1, H)), _wspec((H, H)), _wspec((1, H)),
            _wspec((H, H)), _wspec((1, H)), _wspec((1, H)), _wspec((1, H)),
        ],
        out_specs=_rspec(RE, H),
        out_shape=jax.ShapeDtypeStruct((E_PAD, H), _f32),
    )(ea, ws[0][0], bs[0], ws[1][0], bs[1], ws[2][0], bs[2], g, bln)


def _tc_decoder(node, w0, b0, w1p, b1p):
    def body(n_r, w0_r, b0_r, w1_r, b1_r, o_r):
        h = jnp.maximum(_dot(n_r[...], w0_r[...]) + b0_r[...], 0.0)
        o_r[...] = _dot(h, w1_r[...]) + b1_r[...]

    return pl.pallas_call(
        body,
        grid=(N // RN,),
        in_specs=[_rspec(RN, H), _wspec((H, H)), _wspec((1, H)),
                  _wspec((H, H)), _wspec((1, H))],
        out_specs=_rspec(RN, H),
        out_shape=jax.ShapeDtypeStruct((N, H), _f32),
    )(node, w0, b0, w1p, b1p)


def kernel(x, edge_index, edge_attr, params):
    src = edge_index[0].astype(jnp.int32)
    dst = edge_index[1].astype(jnp.int32)
    E = src.shape[0]
    pad = E_PAD - E

    src_p = jnp.concatenate([src, jnp.zeros((pad,), jnp.int32)])
    dst_g = jnp.concatenate([dst, jnp.zeros((pad,), jnp.int32)])
    dst_s = jnp.concatenate([dst, jnp.full((pad,), N, jnp.int32)])
    ea_p = jnp.concatenate(
        [edge_attr, jnp.zeros((pad, edge_attr.shape[1]), edge_attr.dtype)])
    si = src_p.reshape(E_PAD // CHUNK, CHUNK)
    di_g = dst_g.reshape(E_PAD // CHUNK, CHUNK)
    di_s = dst_s.reshape(E_PAD // CHUNK, CHUNK)

    r = lambda v: v.reshape(1, H)
    blocks = params["blocks"]

    def eb_split(blk):
        (w1, b1), (w2, b2), (w3, b3) = blk["eb_mlp"]
        g, bln = blk["eb_ln"]
        return (w1[:H], w1[H:2 * H], w1[2 * H:], r(b1), w2, r(b2), w3, r(b3),
                r(g), r(bln))

    enc_n_ws = params["enc_node_mlp"]
    enc_n_bs = [r(b) for (_, b) in enc_n_ws]
    enc_e_ws = params["enc_edge_mlp"]
    enc_e_bs = [r(b) for (_, b) in enc_e_ws]
    g_n, b_n = params["enc_node_ln"]
    g_e, b_e = params["enc_edge_ln"]

    ws1, wd1 = blocks[0]["eb_mlp"][0][0][:H], blocks[0]["eb_mlp"][0][0][H:2 * H]
    node, a_tab, b_tab = _tc_enc_node(
        x, enc_n_ws, enc_n_bs, r(g_n), r(b_n), ws1, wd1)
    edge = _tc_enc_edge(ea_p, enc_e_ws, enc_e_bs, r(g_e), r(b_e))

    s_prev = _sc_scatter(edge, di_s)

    for k, blk in enumerate(blocks):
        w1s, w1d, w1e, b1, w2, b2, w3, b3, ge, bline = eb_split(blk)
        ga, gb = _sc_gather(a_tab, b_tab, si, di_g)
        edge = _tc_edge_block(ga, gb, edge, w1e, b1, w2, b2, w3, b3, ge, bline)
        s_k = _sc_scatter(edge, di_s)

        (nw1, nb1), (nw2, nb2), (nw3, nb3) = blk["nb_mlp"]
        gn, blnn = blk["nb_ln"]
        if k + 1 < len(blocks):
            nws, nwd = (blocks[k + 1]["eb_mlp"][0][0][:H],
                        blocks[k + 1]["eb_mlp"][0][0][H:2 * H])
            node, a_tab, b_tab = _tc_node_block(
                node, s_k, s_prev, nw1[:H], nw1[H:], r(nb1), nw2, r(nb2),
                nw3, r(nb3), r(gn), r(blnn), nws, nwd)
        else:
            node = _tc_node_block(
                node, s_k, s_prev, nw1[:H], nw1[H:], r(nb1), nw2, r(nb2),
                nw3, r(nb3), r(gn), r(blnn), None, None)
        s_prev = s_k

    (dw0, db0), (dw1, db1) = params["dec_mlp"]
    out_dim = dw1.shape[1]
    w1p = jnp.zeros((H, H), _f32).at[:, :out_dim].set(dw1)
    b1p = jnp.zeros((1, H), _f32).at[0, :out_dim].set(db1)
    out = _tc_decoder(node, dw0, r(db0), w1p, b1p)
    return out[:, :out_dim]


# speedup vs baseline: 1.6657x; 1.0554x over previous
import functools

import jax
import jax.numpy as jnp
from jax import lax
from jax.experimental import pallas as pl
from jax.experimental.pallas import tpu as pltpu
from jax.experimental.pallas import tpu_sc as plsc

N = 10000
H = 128
NC = 2
NS = 16
NW = NC * NS
CHUNK = 80
NCHUNK = 4096
E_PAD = NCHUNK * CHUNK
NACC = 10240
NB = 4
RE = 2048
RN = 2000

_f32 = jnp.float32


def _mesh():
    return plsc.VectorSubcoreMesh(
        core_axis_name="c", subcore_axis_name="s", num_cores=NC, num_subcores=NS
    )


_RPS = NCHUNK // NS
_GIT = _RPS // (2 * NB)


@functools.partial(
    pl.kernel,
    out_type=jax.ShapeDtypeStruct((NC, E_PAD, H), _f32),
    mesh=_mesh(),
    scratch_types=(
        [pltpu.VMEM((_RPS, CHUNK), jnp.int32)]
        + [pltpu.VMEM((CHUNK, H), _f32)] * (2 * NB)
        + [pltpu.SemaphoreType.DMA] * (4 * NB)
    ),
)
def _sc_gather(tbl, idx, out, idxb, *sc):
    slots = sc[:2 * NB]
    gsem = sc[2 * NB:4 * NB]
    ssem = sc[4 * NB:6 * NB]
    c = lax.axis_index("c")
    s = lax.axis_index("s")
    base = s * _RPS
    pltpu.sync_copy(idx.at[c, pl.ds(base, _RPS)], idxb)

    def issue_g(off, w, b):
        pltpu.async_copy(tbl.at[idxb.at[w * NB + b]], slots[off + b], gsem[off + b])

    def wait_g(off, b):
        pltpu.make_async_copy(
            tbl.at[pl.ds(0, CHUNK)], slots[off + b], gsem[off + b]).wait()

    def issue_s(off, w, b):
        row = (base + w * NB + b) * CHUNK
        pltpu.async_copy(slots[off + b], out.at[c, pl.ds(row, CHUNK)],
                         ssem[off + b])

    def wait_s(off, b):
        pltpu.make_async_copy(
            slots[off + b], out.at[c, pl.ds(0, CHUNK)], ssem[off + b]).wait()

    for b in range(NB):
        issue_g(0, 0, b)
    for b in range(NB):
        issue_g(NB, 1, b)

    def body(g, carry):
        wa = 2 * g
        for b in range(NB):
            wait_g(0, b)
            issue_s(0, wa, b)
        for b in range(NB):
            wait_s(0, b)

        @pl.when(g < _GIT - 1)
        def _():
            for b in range(NB):
                issue_g(0, wa + 2, b)

        for b in range(NB):
            wait_g(NB, b)
            issue_s(NB, wa + 1, b)
        for b in range(NB):
            wait_s(NB, b)

        @pl.when(g < _GIT - 1)
        def _():
            for b in range(NB):
                issue_g(NB, wa + 3, b)

        return carry

    lax.fori_loop(0, _GIT, body, 0)


SCH = 32
SNB = 2
_SCPW = E_PAD // NW // SCH
_SIT = _SCPW // (2 * SNB)


@functools.partial(
    pl.kernel,
    out_type=jax.ShapeDtypeStruct((NC, NACC, H), _f32),
    mesh=_mesh(),
    scratch_types=[
        pltpu.VMEM((_SCPW, SCH), jnp.int32),
        pltpu.VMEM((SCH, H), _f32),
        pltpu.VMEM_SHARED((NACC, H), _f32),
    ],
)
def _sc_scatter_simple(e_hbm, di_hbm, out_hbm, didx, rows, acc):
    c = lax.axis_index("c")
    s = lax.axis_index("s")
    wid = s * NC + c
    base = wid * _SCPW

    def zrow(i, carry):
        def zcol(k, carry2):
            rows[i, pl.ds(k * 16, 16)] = jnp.zeros((16,), _f32)
            return carry2
        return lax.fori_loop(0, H // 16, zcol, carry)

    lax.fori_loop(0, SCH, zrow, 0)
    rpw = NACC // NS
    for t in range(rpw // SCH):
        pltpu.sync_copy(rows, acc.at[pl.ds(s * rpw + t * SCH, SCH)])
    plsc.subcore_barrier()

    pltpu.sync_copy(di_hbm.at[pl.ds(base, _SCPW)], didx)

    def step(j, carry):
        row0 = (base + j) * SCH
        pltpu.sync_copy(e_hbm.at[pl.ds(row0, SCH)], rows)
        pltpu.sync_copy(rows, acc.at[didx.at[j]], add=True)
        return carry

    lax.fori_loop(0, _SCPW, step, 0)
    plsc.subcore_barrier()
    pltpu.sync_copy(acc.at[pl.ds(s * rpw, rpw)],
                    out_hbm.at[c, pl.ds(s * rpw, rpw)])


@functools.partial(
    pl.kernel,
    out_type=jax.ShapeDtypeStruct((NC, NACC, H), _f32),
    mesh=_mesh(),
    scratch_types=(
        [pltpu.VMEM((SNB, SCH), jnp.int32)] * 2
        + [pltpu.VMEM((SCH, H), _f32)] * (2 * SNB)
        + [pltpu.SemaphoreType.DMA] * (4 * SNB)
        + [pltpu.SemaphoreType.DMA] * 2
        + [pltpu.VMEM_SHARED((NACC, H), _f32)]
    ),
)
def _sc_scatter(e_hbm, di_hbm, out_hbm, *sc):
    idxb = sc[:2]
    slots = sc[2:2 + 2 * SNB]
    lsem = sc[2 + 2 * SNB:2 + 4 * SNB]
    ssem = sc[2 + 4 * SNB:2 + 6 * SNB]
    ixsem = sc[2 + 6 * SNB:4 + 6 * SNB]
    acc = sc[4 + 6 * SNB]
    c = lax.axis_index("c")
    s = lax.axis_index("s")
    wid = s * NC + c
    base = wid * _SCPW

    def zrow(i, carry):
        def zcol(k, carry2):
            slots[0][i, pl.ds(k * 16, 16)] = jnp.zeros((16,), _f32)
            return carry2
        return lax.fori_loop(0, H // 16, zcol, carry)

    lax.fori_loop(0, SCH, zrow, 0)
    rpw = NACC // NS
    for t in range(rpw // SCH):
        pltpu.sync_copy(slots[0], acc.at[pl.ds(s * rpw + t * SCH, SCH)])
    plsc.subcore_barrier()

    def issue_ix(hb, w):
        pltpu.async_copy(di_hbm.at[pl.ds(base + w * SNB, SNB)], idxb[hb],
                         ixsem[hb])

    def wait_ix(hb):
        pltpu.make_async_copy(
            di_hbm.at[pl.ds(base, SNB)], idxb[hb], ixsem[hb]).wait()

    def issue_l(off, w, b):
        row = (base + w * SNB + b) * SCH
        pltpu.async_copy(e_hbm.at[pl.ds(row, SCH)], slots[off + b],
                         lsem[off + b])

    def wait_l(off, b):
        pltpu.make_async_copy(
            e_hbm.at[pl.ds(0, SCH)], slots[off + b], lsem[off + b]).wait()

    def issue_sc(hb, off, b):
        pltpu.async_copy(slots[off + b], acc.at[idxb[hb].at[b]],
                         ssem[off + b], add=True)

    def wait_sc(hb, off, b):
        pltpu.make_async_copy(
            slots[off + b], acc.at[idxb[hb].at[b]], ssem[off + b]).wait()

    issue_ix(0, 0)
    issue_ix(1, 1)
    for b in range(SNB):
        issue_l(0, 0, b)
    for b in range(SNB):
        issue_l(SNB, 1, b)

    def body(g, carry):
        wa = 2 * g
        wait_ix(0)
        for b in range(SNB):
            wait_l(0, b)
            issue_sc(0, 0, b)
            wait_sc(0, 0, b)

        @pl.when(g < _SIT - 1)
        def _():
            issue_ix(0, wa + 2)
            for b in range(SNB):
                issue_l(0, wa + 2, b)

        wait_ix(1)
        for b in range(SNB):
            wait_l(SNB, b)
            issue_sc(1, SNB, b)
            wait_sc(1, SNB, b)

        @pl.when(g < _SIT - 1)
        def _():
            issue_ix(1, wa + 3)
            for b in range(SNB):
                issue_l(SNB, wa + 3, b)

        return carry

    lax.fori_loop(0, _SIT, body, 0)
    plsc.subcore_barrier()
    pltpu.sync_copy(acc.at[pl.ds(s * rpw, rpw)],
                    out_hbm.at[c, pl.ds(s * rpw, rpw)])


def _dot(x, w):
    return jnp.dot(x, w, preferred_element_type=_f32,
                   precision=lax.Precision.HIGHEST)


def _ln(h, g, b):
    mu = jnp.mean(h, axis=-1, keepdims=True)
    var = jnp.mean((h - mu) * (h - mu), axis=-1, keepdims=True)
    return (h - mu) * lax.rsqrt(var + 1e-5) * g + b


def _wspec(shape):
    return pl.BlockSpec(shape, lambda i: (0,) * len(shape))


def _rspec(rows, cols):
    return pl.BlockSpec((rows, cols), lambda i: (i, 0))


def _tc_edge_block(g2, e, w1e, b1, w2, b2, w3, b3, g, bln):
    def body(g2_r, e_r, w1e_r, b1_r, w2_r, b2_r, w3_r, b3_r, g_r, bln_r, o_r):
        ev = e_r[...]
        h = jnp.maximum(g2_r[0] + g2_r[1] + _dot(ev, w1e_r[...]) + b1_r[...], 0.0)
        h = jnp.maximum(_dot(h, w2_r[...]) + b2_r[...], 0.0)
        h = _dot(h, w3_r[...]) + b3_r[...]
        o_r[...] = ev + _ln(h, g_r[...], bln_r[...])

    return pl.pallas_call(
        body,
        grid=(E_PAD // RE,),
        in_specs=[
            pl.BlockSpec((NC, RE, H), lambda i: (0, i, 0)), _rspec(RE, H),
            _wspec((H, H)), _wspec((1, H)), _wspec((H, H)), _wspec((1, H)),
            _wspec((H, H)), _wspec((1, H)), _wspec((1, H)), _wspec((1, H)),
        ],
        out_specs=_rspec(RE, H),
        out_shape=jax.ShapeDtypeStruct((E_PAD, H), _f32),
    )(g2, e, w1e, b1, w2, b2, w3, b3, g, bln)


def _tc_node_block(node, sk, sp, w1n, w1a, b1, w2, b2, w3, b3, g, bln, wsn, wdn):
    have_next = wsn is not None

    def body(n_r, sk_r, sp_r, w1n_r, w1a_r, b1_r, w2_r, b2_r, w3_r, b3_r,
             g_r, bln_r, *rest):
        if have_next:
            wsn_r, wdn_r, o_r, ab_r = rest
        else:
            (o_r,) = rest
        nv = n_r[...]
        agg = (sk_r[0] + sk_r[1]) - (sp_r[0] + sp_r[1])
        h = jnp.maximum(_dot(nv, w1n_r[...]) + _dot(agg, w1a_r[...]) + b1_r[...], 0.0)
        h = jnp.maximum(_dot(h, w2_r[...]) + b2_r[...], 0.0)
        h = _dot(h, w3_r[...]) + b3_r[...]
        nn = nv + _ln(h, g_r[...], bln_r[...])
        o_r[...] = nn
        if have_next:
            ab_r[0] = _dot(nn, wsn_r[...])
            ab_r[1] = _dot(nn, wdn_r[...])

    sspec = pl.BlockSpec((NC, RN, H), lambda i: (0, i, 0))
    in_specs = [
        _rspec(RN, H), sspec, sspec,
        _wspec((H, H)), _wspec((H, H)), _wspec((1, H)), _wspec((H, H)),
        _wspec((1, H)), _wspec((H, H)), _wspec((1, H)), _wspec((1, H)),
        _wspec((1, H)),
    ]
    args = [node, sk, sp, w1n, w1a, b1, w2, b2, w3, b3, g, bln]
    nshape = jax.ShapeDtypeStruct((N, H), _f32)
    if have_next:
        in_specs += [_wspec((H, H)), _wspec((H, H))]
        args += [wsn, wdn]
        out_specs = (_rspec(RN, H), pl.BlockSpec((NC, RN, H), lambda i: (0, i, 0)))
        out_shape = (nshape, jax.ShapeDtypeStruct((NC, N, H), _f32))
    else:
        out_specs = _rspec(RN, H)
        out_shape = nshape
    return pl.pallas_call(
        body,
        grid=(N // RN,),
        in_specs=in_specs,
        out_specs=out_specs,
        out_shape=out_shape,
    )(*args)


def _tc_enc_node(x, ws, bs, g, bln, wsn, wdn):
    def body(x_r, w0, b0, w1, b1, w2, b2, g_r, bln_r, wsn_r, wdn_r,
             o_r, ab_r):
        h = jnp.maximum(_dot(x_r[...], w0[...]) + b0[...], 0.0)
        h = jnp.maximum(_dot(h, w1[...]) + b1[...], 0.0)
        h = _dot(h, w2[...]) + b2[...]
        nn = _ln(h, g_r[...], bln_r[...])
        o_r[...] = nn
        ab_r[0] = _dot(nn, wsn_r[...])
        ab_r[1] = _dot(nn, wdn_r[...])

    nshape = jax.ShapeDtypeStruct((N, H), _f32)
    return pl.pallas_call(
        body,
        grid=(N // RN,),
        in_specs=[_rspec(RN, H)] + [
            _wspec((H, H)), _wspec((1, H)), _wspec((H, H)), _wspec((1, H)),
            _wspec((H, H)), _wspec((1, H)), _wspec((1, H)), _wspec((1, H)),
            _wspec((H, H)), _wspec((H, H)),
        ],
        out_specs=(_rspec(RN, H), pl.BlockSpec((NC, RN, H), lambda i: (0, i, 0))),
        out_shape=(nshape, jax.ShapeDtypeStruct((NC, N, H), _f32)),
    )(x, ws[0][0], bs[0], ws[1][0], bs[1], ws[2][0], bs[2], g, bln, wsn, wdn)


def _tc_enc_edge(ea, ws, bs, g, bln):
    ein = ea.shape[1]

    def body(e_r, w0, b0, w1, b1, w2, b2, g_r, bln_r, o_r):
        h = jnp.maximum(_dot(e_r[...], w0[...]) + b0[...], 0.0)
        h = jnp.maximum(_dot(h, w1[...]) + b1[...], 0.0)
        h = _dot(h, w2[...]) + b2[...]
        o_r[...] = _ln(h, g_r[...], bln_r[...])

    return pl.pallas_call(
        body,
        grid=(E_PAD // RE,),
        in_specs=[_rspec(RE, ein)] + [
            _wspec((ein, H)), _wspec((1, H)), _wspec((H, H)), _wspec((1, H)),
            _wspec((H, H)), _wspec((1, H)), _wspec((1, H)), _wspec((1, H)),
        ],
        out_specs=_rspec(RE, H),
        out_shape=jax.ShapeDtypeStruct((E_PAD, H), _f32),
    )(ea, ws[0][0], bs[0], ws[1][0], bs[1], ws[2][0], bs[2], g, bln)


def _tc_decoder(node, w0, b0, w1p, b1p):
    def body(n_r, w0_r, b0_r, w1_r, b1_r, o_r):
        h = jnp.maximum(_dot(n_r[...], w0_r[...]) + b0_r[...], 0.0)
        o_r[...] = _dot(h, w1_r[...]) + b1_r[...]

    return pl.pallas_call(
        body,
        grid=(N // RN,),
        in_specs=[_rspec(RN, H), _wspec((H, H)), _wspec((1, H)),
                  _wspec((H, H)), _wspec((1, H))],
        out_specs=_rspec(RN, H),
        out_shape=jax.ShapeDtypeStruct((N, H), _f32),
    )(node, w0, b0, w1p, b1p)


def kernel(x, edge_index, edge_attr, params):
    src = edge_index[0].astype(jnp.int32)
    dst = edge_index[1].astype(jnp.int32)
    E = src.shape[0]
    pad = E_PAD - E

    src_p = jnp.concatenate([src, jnp.zeros((pad,), jnp.int32)])
    dst_g = jnp.concatenate([dst, jnp.zeros((pad,), jnp.int32)])
    dst_s = jnp.concatenate([dst, jnp.full((pad,), N, jnp.int32)])
    ea_p = jnp.concatenate(
        [edge_attr, jnp.zeros((pad, edge_attr.shape[1]), edge_attr.dtype)])
    idx_g = jnp.stack([src_p.reshape(NCHUNK, CHUNK),
                       dst_g.reshape(NCHUNK, CHUNK) + N])
    di_s = dst_s.reshape(E_PAD // SCH, SCH)

    r = lambda v: v.reshape(1, H)
    blocks = params["blocks"]

    def eb_split(blk):
        (w1, b1), (w2, b2), (w3, b3) = blk["eb_mlp"]
        g, bln = blk["eb_ln"]
        return (w1[:H], w1[H:2 * H], w1[2 * H:], r(b1), w2, r(b2), w3, r(b3),
                r(g), r(bln))

    enc_n_ws = params["enc_node_mlp"]
    enc_n_bs = [r(b) for (_, b) in enc_n_ws]
    enc_e_ws = params["enc_edge_mlp"]
    enc_e_bs = [r(b) for (_, b) in enc_e_ws]
    g_n, b_n = params["enc_node_ln"]
    g_e, b_e = params["enc_edge_ln"]

    ws1, wd1 = blocks[0]["eb_mlp"][0][0][:H], blocks[0]["eb_mlp"][0][0][H:2 * H]
    node, ab_tab = _tc_enc_node(
        x, enc_n_ws, enc_n_bs, r(g_n), r(b_n), ws1, wd1)
    edge = _tc_enc_edge(ea_p, enc_e_ws, enc_e_bs, r(g_e), r(b_e))

    s_prev = _sc_scatter(edge, di_s)

    for k, blk in enumerate(blocks):
        w1s, w1d, w1e, b1, w2, b2, w3, b3, ge, bline = eb_split(blk)
        g2 = _sc_gather(ab_tab.reshape(2 * N, H), idx_g)
        edge = _tc_edge_block(g2, edge, w1e, b1, w2, b2, w3, b3, ge, bline)
        s_k = _sc_scatter(edge, di_s)

        (nw1, nb1), (nw2, nb2), (nw3, nb3) = blk["nb_mlp"]
        gn, blnn = blk["nb_ln"]
        if k + 1 < len(blocks):
            nws, nwd = (blocks[k + 1]["eb_mlp"][0][0][:H],
                        blocks[k + 1]["eb_mlp"][0][0][H:2 * H])
            node, ab_tab = _tc_node_block(
                node, s_k, s_prev, nw1[:H], nw1[H:], r(nb1), nw2, r(nb2),
                nw3, r(nb3), r(gn), r(blnn), nws, nwd)
        else:
            node = _tc_node_block(
                node, s_k, s_prev, nw1[:H], nw1[H:], r(nb1), nw2, r(nb2),
                nw3, r(nb3), r(gn), r(blnn), None, None)
        s_prev = s_k

    (dw0, db0), (dw1, db1) = params["dec_mlp"]
    out_dim = dw1.shape[1]
    w1p = jnp.zeros((H, H), _f32).at[:, :out_dim].set(dw1)
    b1p = jnp.zeros((1, H), _f32).at[0, :out_dim].set(db1)
    out = _tc_decoder(node, dw0, r(db0), w1p, b1p)
    return out[:, :out_dim]


# speedup vs baseline: 1.7505x; 1.0509x over previous
import functools

import jax
import jax.numpy as jnp
from jax import lax
from jax.experimental import pallas as pl
from jax.experimental.pallas import tpu as pltpu
from jax.experimental.pallas import tpu_sc as plsc

N = 10000
H = 128
NC = 2
NS = 16
NW = NC * NS
CHUNK = 128
NCHUNK = 2560
E_PAD = NCHUNK * CHUNK
NACC = 10240
NB = 2
RE = 2048
RN = 2000

_f32 = jnp.float32


def _mesh():
    return plsc.VectorSubcoreMesh(
        core_axis_name="c", subcore_axis_name="s", num_cores=NC, num_subcores=NS
    )


_RPS = NCHUNK // NS
_GIT = _RPS // (2 * NB)


@functools.partial(
    pl.kernel,
    out_type=jax.ShapeDtypeStruct((NC, E_PAD, H), _f32),
    mesh=_mesh(),
    scratch_types=(
        [pltpu.VMEM((_RPS, CHUNK), jnp.int32)]
        + [pltpu.VMEM((CHUNK, H), _f32)] * (2 * NB)
        + [pltpu.SemaphoreType.DMA] * (4 * NB)
    ),
)
def _sc_gather(tbl, idx, out, idxb, *sc):
    slots = sc[:2 * NB]
    gsem = sc[2 * NB:4 * NB]
    ssem = sc[4 * NB:6 * NB]
    c = lax.axis_index("c")
    s = lax.axis_index("s")
    base = s * _RPS
    pltpu.sync_copy(idx.at[c, pl.ds(base, _RPS)], idxb)

    def issue_g(off, w, b):
        pltpu.async_copy(tbl.at[idxb.at[w * NB + b]], slots[off + b], gsem[off + b])

    def wait_g(off, b):
        pltpu.make_async_copy(
            tbl.at[pl.ds(0, CHUNK)], slots[off + b], gsem[off + b]).wait()

    def issue_s(off, w, b):
        row = (base + w * NB + b) * CHUNK
        pltpu.async_copy(slots[off + b], out.at[c, pl.ds(row, CHUNK)],
                         ssem[off + b])

    def wait_s(off, b):
        pltpu.make_async_copy(
            slots[off + b], out.at[c, pl.ds(0, CHUNK)], ssem[off + b]).wait()

    for b in range(NB):
        issue_g(0, 0, b)
    for b in range(NB):
        issue_g(NB, 1, b)

    def body(g, carry):
        wa = 2 * g
        for b in range(NB):
            wait_g(0, b)
            issue_s(0, wa, b)
        for b in range(NB):
            wait_s(0, b)

        @pl.when(g < _GIT - 1)
        def _():
            for b in range(NB):
                issue_g(0, wa + 2, b)

        for b in range(NB):
            wait_g(NB, b)
            issue_s(NB, wa + 1, b)
        for b in range(NB):
            wait_s(NB, b)

        @pl.when(g < _GIT - 1)
        def _():
            for b in range(NB):
                issue_g(NB, wa + 3, b)

        return carry

    lax.fori_loop(0, _GIT, body, 0)


SCH = 80
SNB = 1
_SCPW = E_PAD // NW // SCH
_SIT = _SCPW // (2 * SNB)


@functools.partial(
    pl.kernel,
    out_type=jax.ShapeDtypeStruct((NC, NACC, H), _f32),
    mesh=_mesh(),
    scratch_types=[
        pltpu.VMEM((_SCPW, SCH), jnp.int32),
        pltpu.VMEM((SCH, H), _f32),
        pltpu.VMEM_SHARED((NACC, H), _f32),
    ],
)
def _sc_scatter_simple(e_hbm, di_hbm, out_hbm, didx, rows, acc):
    c = lax.axis_index("c")
    s = lax.axis_index("s")
    wid = s * NC + c
    base = wid * _SCPW

    def zrow(i, carry):
        def zcol(k, carry2):
            rows[i, pl.ds(k * 16, 16)] = jnp.zeros((16,), _f32)
            return carry2
        return lax.fori_loop(0, H // 16, zcol, carry)

    lax.fori_loop(0, SCH, zrow, 0)
    rpw = NACC // NS
    for t in range(rpw // SCH):
        pltpu.sync_copy(rows, acc.at[pl.ds(s * rpw + t * SCH, SCH)])
    plsc.subcore_barrier()

    pltpu.sync_copy(di_hbm.at[pl.ds(base, _SCPW)], didx)

    def step(j, carry):
        row0 = (base + j) * SCH
        pltpu.sync_copy(e_hbm.at[pl.ds(row0, SCH)], rows)
        pltpu.sync_copy(rows, acc.at[didx.at[j]], add=True)
        return carry

    lax.fori_loop(0, _SCPW, step, 0)
    plsc.subcore_barrier()
    pltpu.sync_copy(acc.at[pl.ds(s * rpw, rpw)],
                    out_hbm.at[c, pl.ds(s * rpw, rpw)])


@functools.partial(
    pl.kernel,
    out_type=jax.ShapeDtypeStruct((NC, NACC, H), _f32),
    mesh=_mesh(),
    scratch_types=(
        [pltpu.VMEM((SNB, SCH), jnp.int32)] * 2
        + [pltpu.VMEM((SCH, H), _f32)] * (2 * SNB)
        + [pltpu.SemaphoreType.DMA] * (4 * SNB)
        + [pltpu.SemaphoreType.DMA] * 2
        + [pltpu.VMEM_SHARED((NACC, H), _f32)]
    ),
)
def _sc_scatter(e_hbm, di_hbm, out_hbm, *sc):
    idxb = sc[:2]
    slots = sc[2:2 + 2 * SNB]
    lsem = sc[2 + 2 * SNB:2 + 4 * SNB]
    ssem = sc[2 + 4 * SNB:2 + 6 * SNB]
    ixsem = sc[2 + 6 * SNB:4 + 6 * SNB]
    acc = sc[4 + 6 * SNB]
    c = lax.axis_index("c")
    s = lax.axis_index("s")
    wid = s * NC + c
    base = wid * _SCPW

    def zrow(i, carry):
        def zcol(k, carry2):
            slots[0][i, pl.ds(k * 16, 16)] = jnp.zeros((16,), _f32)
            return carry2
        return lax.fori_loop(0, H // 16, zcol, carry)

    lax.fori_loop(0, SCH, zrow, 0)
    rpw = NACC // NS
    for t in range(rpw // SCH):
        pltpu.sync_copy(slots[0], acc.at[pl.ds(s * rpw + t * SCH, SCH)])
    rem = rpw % SCH
    if rem:
        pltpu.sync_copy(slots[0].at[pl.ds(0, rem)],
                        acc.at[pl.ds(s * rpw + rpw - rem, rem)])
    plsc.subcore_barrier()

    def issue_ix(hb, w):
        pltpu.async_copy(di_hbm.at[pl.ds(base + w * SNB, SNB)], idxb[hb],
                         ixsem[hb])

    def wait_ix(hb):
        pltpu.make_async_copy(
            di_hbm.at[pl.ds(base, SNB)], idxb[hb], ixsem[hb]).wait()

    def issue_l(off, w, b):
        row = (base + w * SNB + b) * SCH
        pltpu.async_copy(e_hbm.at[pl.ds(row, SCH)], slots[off + b],
                         lsem[off + b])

    def wait_l(off, b):
        pltpu.make_async_copy(
            e_hbm.at[pl.ds(0, SCH)], slots[off + b], lsem[off + b]).wait()

    def issue_sc(hb, off, b):
        pltpu.async_copy(slots[off + b], acc.at[idxb[hb].at[b]],
                         ssem[off + b], add=True)

    def wait_sc(hb, off, b):
        pltpu.make_async_copy(
            slots[off + b], acc.at[idxb[hb].at[b]], ssem[off + b]).wait()

    issue_ix(0, 0)
    issue_ix(1, 1)
    for b in range(SNB):
        issue_l(0, 0, b)
    for b in range(SNB):
        issue_l(SNB, 1, b)

    def body(g, carry):
        wa = 2 * g
        wait_ix(0)
        for b in range(SNB):
            wait_l(0, b)
            issue_sc(0, 0, b)
            wait_sc(0, 0, b)

        @pl.when(g < _SIT - 1)
        def _():
            issue_ix(0, wa + 2)
            for b in range(SNB):
                issue_l(0, wa + 2, b)

        wait_ix(1)
        for b in range(SNB):
            wait_l(SNB, b)
            issue_sc(1, SNB, b)
            wait_sc(1, SNB, b)

        @pl.when(g < _SIT - 1)
        def _():
            issue_ix(1, wa + 3)
            for b in range(SNB):
                issue_l(SNB, wa + 3, b)

        return carry

    lax.fori_loop(0, _SIT, body, 0)
    plsc.subcore_barrier()
    pltpu.sync_copy(acc.at[pl.ds(s * rpw, rpw)],
                    out_hbm.at[c, pl.ds(s * rpw, rpw)])


def _dot(x, w):
    return jnp.dot(x, w, preferred_element_type=_f32,
                   precision=lax.Precision.HIGHEST)


def _ln(h, g, b):
    mu = jnp.mean(h, axis=-1, keepdims=True)
    var = jnp.mean((h - mu) * (h - mu), axis=-1, keepdims=True)
    return (h - mu) * lax.rsqrt(var + 1e-5) * g + b


def _wspec(shape):
    return pl.BlockSpec(shape, lambda i: (0,) * len(shape))


def _rspec(rows, cols):
    return pl.BlockSpec((rows, cols), lambda i: (i, 0))


def _tc_edge_block(g2, e, w1e, b1, w2, b2, w3, b3, g, bln):
    def body(g2_r, e_r, w1e_r, b1_r, w2_r, b2_r, w3_r, b3_r, g_r, bln_r, o_r):
        ev = e_r[...]
        h = jnp.maximum(g2_r[0] + g2_r[1] + _dot(ev, w1e_r[...]) + b1_r[...], 0.0)
        h = jnp.maximum(_dot(h, w2_r[...]) + b2_r[...], 0.0)
        h = _dot(h, w3_r[...]) + b3_r[...]
        o_r[...] = ev + _ln(h, g_r[...], bln_r[...])

    return pl.pallas_call(
        body,
        grid=(E_PAD // RE,),
        in_specs=[
            pl.BlockSpec((NC, RE, H), lambda i: (0, i, 0)), _rspec(RE, H),
            _wspec((H, H)), _wspec((1, H)), _wspec((H, H)), _wspec((1, H)),
            _wspec((H, H)), _wspec((1, H)), _wspec((1, H)), _wspec((1, H)),
        ],
        out_specs=_rspec(RE, H),
        out_shape=jax.ShapeDtypeStruct((E_PAD, H), _f32),
    )(g2, e, w1e, b1, w2, b2, w3, b3, g, bln)


def _tc_node_block(node, sk, sp, w1n, w1a, b1, w2, b2, w3, b3, g, bln, wsn, wdn):
    have_next = wsn is not None

    def body(n_r, sk_r, sp_r, w1n_r, w1a_r, b1_r, w2_r, b2_r, w3_r, b3_r,
             g_r, bln_r, *rest):
        if have_next:
            wsn_r, wdn_r, o_r, ab_r = rest
        else:
            (o_r,) = rest
        nv = n_r[...]
        agg = (sk_r[0] + sk_r[1]) - (sp_r[0] + sp_r[1])
        h = jnp.maximum(_dot(nv, w1n_r[...]) + _dot(agg, w1a_r[...]) + b1_r[...], 0.0)
        h = jnp.maximum(_dot(h, w2_r[...]) + b2_r[...], 0.0)
        h = _dot(h, w3_r[...]) + b3_r[...]
        nn = nv + _ln(h, g_r[...], bln_r[...])
        o_r[...] = nn
        if have_next:
            ab_r[0] = _dot(nn, wsn_r[...])
            ab_r[1] = _dot(nn, wdn_r[...])

    sspec = pl.BlockSpec((NC, RN, H), lambda i: (0, i, 0))
    in_specs = [
        _rspec(RN, H), sspec, sspec,
        _wspec((H, H)), _wspec((H, H)), _wspec((1, H)), _wspec((H, H)),
        _wspec((1, H)), _wspec((H, H)), _wspec((1, H)), _wspec((1, H)),
        _wspec((1, H)),
    ]
    args = [node, sk, sp, w1n, w1a, b1, w2, b2, w3, b3, g, bln]
    nshape = jax.ShapeDtypeStruct((N, H), _f32)
    if have_next:
        in_specs += [_wspec((H, H)), _wspec((H, H))]
        args += [wsn, wdn]
        out_specs = (_rspec(RN, H), pl.BlockSpec((NC, RN, H), lambda i: (0, i, 0)))
        out_shape = (nshape, jax.ShapeDtypeStruct((NC, N, H), _f32))
    else:
        out_specs = _rspec(RN, H)
        out_shape = nshape
    return pl.pallas_call(
        body,
        grid=(N // RN,),
        in_specs=in_specs,
        out_specs=out_specs,
        out_shape=out_shape,
    )(*args)


def _tc_enc_node(x, ws, bs, g, bln, wsn, wdn):
    def body(x_r, w0, b0, w1, b1, w2, b2, g_r, bln_r, wsn_r, wdn_r,
             o_r, ab_r):
        h = jnp.maximum(_dot(x_r[...], w0[...]) + b0[...], 0.0)
        h = jnp.maximum(_dot(h, w1[...]) + b1[...], 0.0)
        h = _dot(h, w2[...]) + b2[...]
        nn = _ln(h, g_r[...], bln_r[...])
        o_r[...] = nn
        ab_r[0] = _dot(nn, wsn_r[...])
        ab_r[1] = _dot(nn, wdn_r[...])

    nshape = jax.ShapeDtypeStruct((N, H), _f32)
    return pl.pallas_call(
        body,
        grid=(N // RN,),
        in_specs=[_rspec(RN, H)] + [
            _wspec((H, H)), _wspec((1, H)), _wspec((H, H)), _wspec((1, H)),
            _wspec((H, H)), _wspec((1, H)), _wspec((1, H)), _wspec((1, H)),
            _wspec((H, H)), _wspec((H, H)),
        ],
        out_specs=(_rspec(RN, H), pl.BlockSpec((NC, RN, H), lambda i: (0, i, 0))),
        out_shape=(nshape, jax.ShapeDtypeStruct((NC, N, H), _f32)),
    )(x, ws[0][0], bs[0], ws[1][0], bs[1], ws[2][0], bs[2], g, bln, wsn, wdn)


def _tc_enc_edge(ea, ws, bs, g, bln):
    ein = ea.shape[1]

    def body(e_r, w0, b0, w1, b1, w2, b2, g_r, bln_r, o_r):
        h = jnp.maximum(_dot(e_r[...], w0[...]) + b0[...], 0.0)
        h = jnp.maximum(_dot(h, w1[...]) + b1[...], 0.0)
        h = _dot(h, w2[...]) + b2[...]
        o_r[...] = _ln(h, g_r[...], bln_r[...])

    return pl.pallas_call(
        body,
        grid=(E_PAD // RE,),
        in_specs=[_rspec(RE, ein)] + [
            _wspec((ein, H)), _wspec((1, H)), _wspec((H, H)), _wspec((1, H)),
            _wspec((H, H)), _wspec((1, H)), _wspec((1, H)), _wspec((1, H)),
        ],
        out_specs=_rspec(RE, H),
        out_shape=jax.ShapeDtypeStruct((E_PAD, H), _f32),
    )(ea, ws[0][0], bs[0], ws[1][0], bs[1], ws[2][0], bs[2], g, bln)


def _tc_decoder(node, w0, b0, w1p, b1p):
    def body(n_r, w0_r, b0_r, w1_r, b1_r, o_r):
        h = jnp.maximum(_dot(n_r[...], w0_r[...]) + b0_r[...], 0.0)
        o_r[...] = _dot(h, w1_r[...]) + b1_r[...]

    return pl.pallas_call(
        body,
        grid=(N // RN,),
        in_specs=[_rspec(RN, H), _wspec((H, H)), _wspec((1, H)),
                  _wspec((H, H)), _wspec((1, H))],
        out_specs=_rspec(RN, H),
        out_shape=jax.ShapeDtypeStruct((N, H), _f32),
    )(node, w0, b0, w1p, b1p)


def kernel(x, edge_index, edge_attr, params):
    src = edge_index[0].astype(jnp.int32)
    dst = edge_index[1].astype(jnp.int32)
    E = src.shape[0]
    pad = E_PAD - E

    src_p = jnp.concatenate([src, jnp.zeros((pad,), jnp.int32)])
    dst_g = jnp.concatenate([dst, jnp.zeros((pad,), jnp.int32)])
    dst_s = jnp.concatenate([dst, jnp.full((pad,), N, jnp.int32)])
    ea_p = jnp.concatenate(
        [edge_attr, jnp.zeros((pad, edge_attr.shape[1]), edge_attr.dtype)])
    idx_g = jnp.stack([src_p.reshape(NCHUNK, CHUNK),
                       dst_g.reshape(NCHUNK, CHUNK) + N])
    di_s = dst_s.reshape(E_PAD // SCH, SCH)

    r = lambda v: v.reshape(1, H)
    blocks = params["blocks"]

    def eb_split(blk):
        (w1, b1), (w2, b2), (w3, b3) = blk["eb_mlp"]
        g, bln = blk["eb_ln"]
        return (w1[:H], w1[H:2 * H], w1[2 * H:], r(b1), w2, r(b2), w3, r(b3),
                r(g), r(bln))

    enc_n_ws = params["enc_node_mlp"]
    enc_n_bs = [r(b) for (_, b) in enc_n_ws]
    enc_e_ws = params["enc_edge_mlp"]
    enc_e_bs = [r(b) for (_, b) in enc_e_ws]
    g_n, b_n = params["enc_node_ln"]
    g_e, b_e = params["enc_edge_ln"]

    ws1, wd1 = blocks[0]["eb_mlp"][0][0][:H], blocks[0]["eb_mlp"][0][0][H:2 * H]
    node, ab_tab = _tc_enc_node(
        x, enc_n_ws, enc_n_bs, r(g_n), r(b_n), ws1, wd1)
    edge = _tc_enc_edge(ea_p, enc_e_ws, enc_e_bs, r(g_e), r(b_e))

    s_prev = _sc_scatter(edge, di_s)

    for k, blk in enumerate(blocks):
        w1s, w1d, w1e, b1, w2, b2, w3, b3, ge, bline = eb_split(blk)
        g2 = _sc_gather(ab_tab.reshape(2 * N, H), idx_g)
        edge = _tc_edge_block(g2, edge, w1e, b1, w2, b2, w3, b3, ge, bline)
        s_k = _sc_scatter(edge, di_s)

        (nw1, nb1), (nw2, nb2), (nw3, nb3) = blk["nb_mlp"]
        gn, blnn = blk["nb_ln"]
        if k + 1 < len(blocks):
            nws, nwd = (blocks[k + 1]["eb_mlp"][0][0][:H],
                        blocks[k + 1]["eb_mlp"][0][0][H:2 * H])
            node, ab_tab = _tc_node_block(
                node, s_k, s_prev, nw1[:H], nw1[H:], r(nb1), nw2, r(nb2),
                nw3, r(nb3), r(gn), r(blnn), nws, nwd)
        else:
            node = _tc_node_block(
                node, s_k, s_prev, nw1[:H], nw1[H:], r(nb1), nw2, r(nb2),
                nw3, r(nb3), r(gn), r(blnn), None, None)
        s_prev = s_k

    (dw0, db0), (dw1, db1) = params["dec_mlp"]
    out_dim = dw1.shape[1]
    w1p = jnp.zeros((H, H), _f32).at[:, :out_dim].set(dw1)
    b1p = jnp.zeros((1, H), _f32).at[0, :out_dim].set(db1)
    out = _tc_decoder(node, dw0, r(db0), w1p, b1p)
    return out[:, :out_dim]


# speedup vs baseline: 1.7544x; 1.0023x over previous
import functools

import jax
import jax.numpy as jnp
from jax import lax
from jax.experimental import pallas as pl
from jax.experimental.pallas import tpu as pltpu
from jax.experimental.pallas import tpu_sc as plsc

N = 10000
H = 128
NC = 2
NS = 16
NW = NC * NS
CHUNK = 80
NCHUNK = 4096
E_PAD = NCHUNK * CHUNK
NACC = 10240
NB = 4
RE = 2048
RN = 2000

_f32 = jnp.float32


def _mesh():
    return plsc.VectorSubcoreMesh(
        core_axis_name="c", subcore_axis_name="s", num_cores=NC, num_subcores=NS
    )


_RPS = NCHUNK // NS
_GIT = _RPS // (2 * NB)


@functools.partial(
    pl.kernel,
    out_type=jax.ShapeDtypeStruct((NC, E_PAD, H), _f32),
    mesh=_mesh(),
    scratch_types=(
        [pltpu.VMEM((_RPS, CHUNK), jnp.int32)]
        + [pltpu.VMEM((CHUNK, H), _f32)] * (2 * NB)
        + [pltpu.SemaphoreType.DMA] * (4 * NB)
    ),
)
def _sc_gather(tbl, idx, out, idxb, *sc):
    slots = sc[:2 * NB]
    gsem = sc[2 * NB:4 * NB]
    ssem = sc[4 * NB:6 * NB]
    c = lax.axis_index("c")
    s = lax.axis_index("s")
    base = s * _RPS
    pltpu.sync_copy(idx.at[c, pl.ds(base, _RPS)], idxb)

    def issue_g(off, w, b):
        pltpu.async_copy(tbl.at[idxb.at[w * NB + b]], slots[off + b],
                         gsem[off + b])

    def wait_g(off, b):
        pltpu.make_async_copy(
            tbl.at[pl.ds(0, CHUNK)], slots[off + b], gsem[off + b]).wait()

    def issue_s(off, w, b):
        row = (base + w * NB + b) * CHUNK
        pltpu.async_copy(slots[off + b], out.at[c, pl.ds(row, CHUNK)],
                         ssem[off + b])

    def wait_s(off, b):
        pltpu.make_async_copy(
            slots[off + b], out.at[c, pl.ds(0, CHUNK)], ssem[off + b]).wait()

    for b in range(NB):
        issue_g(0, 0, b)
    for b in range(NB):
        issue_g(NB, 1, b)

    def body(g, carry):
        wa = 2 * g
        for b in range(NB):
            wait_g(0, b)
            issue_s(0, wa, b)
        for b in range(NB):
            wait_s(0, b)

        @pl.when(g < _GIT - 1)
        def _():
            for b in range(NB):
                issue_g(0, wa + 2, b)

        for b in range(NB):
            wait_g(NB, b)
            issue_s(NB, wa + 1, b)
        for b in range(NB):
            wait_s(NB, b)

        @pl.when(g < _GIT - 1)
        def _():
            for b in range(NB):
                issue_g(NB, wa + 3, b)

        return carry

    lax.fori_loop(0, _GIT, body, 0)


SCH = 80
SNB = 1
_SCPW = E_PAD // NW // SCH
_SIT = _SCPW // (2 * SNB)


@functools.partial(
    pl.kernel,
    out_type=jax.ShapeDtypeStruct((NC, NACC, H), _f32),
    mesh=_mesh(),
    scratch_types=[
        pltpu.VMEM((_SCPW, SCH), jnp.int32),
        pltpu.VMEM((SCH, H), _f32),
        pltpu.VMEM_SHARED((NACC, H), _f32),
    ],
)
def _sc_scatter_simple(e_hbm, di_hbm, out_hbm, didx, rows, acc):
    c = lax.axis_index("c")
    s = lax.axis_index("s")
    wid = s * NC + c
    base = wid * _SCPW

    def zrow(i, carry):
        def zcol(k, carry2):
            rows[i, pl.ds(k * 16, 16)] = jnp.zeros((16,), _f32)
            return carry2
        return lax.fori_loop(0, H // 16, zcol, carry)

    lax.fori_loop(0, SCH, zrow, 0)
    rpw = NACC // NS
    for t in range(rpw // SCH):
        pltpu.sync_copy(rows, acc.at[pl.ds(s * rpw + t * SCH, SCH)])
    plsc.subcore_barrier()

    pltpu.sync_copy(di_hbm.at[pl.ds(base, _SCPW)], didx)

    def step(j, carry):
        row0 = (base + j) * SCH
        pltpu.sync_copy(e_hbm.at[pl.ds(row0, SCH)], rows)
        pltpu.sync_copy(rows, acc.at[didx.at[j]], add=True)
        return carry

    lax.fori_loop(0, _SCPW, step, 0)
    plsc.subcore_barrier()
    pltpu.sync_copy(acc.at[pl.ds(s * rpw, rpw)],
                    out_hbm.at[c, pl.ds(s * rpw, rpw)])


@functools.partial(
    pl.kernel,
    out_type=jax.ShapeDtypeStruct((NC, NACC, H), _f32),
    mesh=_mesh(),
    scratch_types=(
        [pltpu.VMEM((SNB, SCH), jnp.int32)] * 2
        + [pltpu.VMEM((SCH, H), _f32)] * (2 * SNB)
        + [pltpu.SemaphoreType.DMA] * (4 * SNB)
        + [pltpu.SemaphoreType.DMA] * 2
        + [pltpu.VMEM_SHARED((NACC, H), _f32)]
    ),
)
def _sc_scatter(e_hbm, di_hbm, out_hbm, *sc):
    idxb = sc[:2]
    slots = sc[2:2 + 2 * SNB]
    lsem = sc[2 + 2 * SNB:2 + 4 * SNB]
    ssem = sc[2 + 4 * SNB:2 + 6 * SNB]
    ixsem = sc[2 + 6 * SNB:4 + 6 * SNB]
    acc = sc[4 + 6 * SNB]
    c = lax.axis_index("c")
    s = lax.axis_index("s")
    wid = s * NC + c
    base = wid * _SCPW

    def zrow(i, carry):
        def zcol(k, carry2):
            slots[0][i, pl.ds(k * 16, 16)] = jnp.zeros((16,), _f32)
            return carry2
        return lax.fori_loop(0, H // 16, zcol, carry)

    lax.fori_loop(0, SCH, zrow, 0)
    rpw = NACC // NS
    for t in range(rpw // SCH):
        pltpu.sync_copy(slots[0], acc.at[pl.ds(s * rpw + t * SCH, SCH)])
    rem = rpw % SCH
    if rem:
        pltpu.sync_copy(slots[0].at[pl.ds(0, rem)],
                        acc.at[pl.ds(s * rpw + rpw - rem, rem)])
    plsc.subcore_barrier()

    def issue_ix(hb, w):
        pltpu.async_copy(di_hbm.at[pl.ds(base + w * SNB, SNB)], idxb[hb],
                         ixsem[hb])

    def wait_ix(hb):
        pltpu.make_async_copy(
            di_hbm.at[pl.ds(base, SNB)], idxb[hb], ixsem[hb]).wait()

    def issue_l(off, w, b):
        row = (base + w * SNB + b) * SCH
        pltpu.async_copy(e_hbm.at[pl.ds(row, SCH)], slots[off + b],
                         lsem[off + b])

    def wait_l(off, b):
        pltpu.make_async_copy(
            e_hbm.at[pl.ds(0, SCH)], slots[off + b], lsem[off + b]).wait()

    def issue_sc(hb, off, b):
        pltpu.async_copy(slots[off + b], acc.at[idxb[hb].at[b]],
                         ssem[off + b], add=True)

    def wait_sc(hb, off, b):
        pltpu.make_async_copy(
            slots[off + b], acc.at[idxb[hb].at[b]], ssem[off + b]).wait()

    issue_ix(0, 0)
    issue_ix(1, 1)
    for b in range(SNB):
        issue_l(0, 0, b)
    for b in range(SNB):
        issue_l(SNB, 1, b)

    def body(g, carry):
        wa = 2 * g
        wait_ix(0)
        for b in range(SNB):
            wait_l(0, b)
            issue_sc(0, 0, b)
            wait_sc(0, 0, b)

        @pl.when(g < _SIT - 1)
        def _():
            issue_ix(0, wa + 2)
            for b in range(SNB):
                issue_l(0, wa + 2, b)

        wait_ix(1)
        for b in range(SNB):
            wait_l(SNB, b)
            issue_sc(1, SNB, b)
            wait_sc(1, SNB, b)

        @pl.when(g < _SIT - 1)
        def _():
            issue_ix(1, wa + 3)
            for b in range(SNB):
                issue_l(SNB, wa + 3, b)

        return carry

    lax.fori_loop(0, _SIT, body, 0)
    plsc.subcore_barrier()
    pltpu.sync_copy(acc.at[pl.ds(s * rpw, rpw)],
                    out_hbm.at[c, pl.ds(s * rpw, rpw)])


def _dot(x, w):
    return jnp.dot(x, w, preferred_element_type=_f32,
                   precision=lax.Precision.HIGHEST)


def _ln(h, g, b):
    mu = jnp.mean(h, axis=-1, keepdims=True)
    var = jnp.mean((h - mu) * (h - mu), axis=-1, keepdims=True)
    return (h - mu) * lax.rsqrt(var + 1e-5) * g + b


def _wspec(shape):
    return pl.BlockSpec(shape, lambda i: (0,) * len(shape))


def _rspec(rows, cols):
    return pl.BlockSpec((rows, cols), lambda i: (i, 0))


def _tc_edge_block(gm, e, w1e, b1, w2, b2, w3, b3, g, bln):
    def body(gm_r, e_r, w1e_r, b1_r, w2_r, b2_r, w3_r, b3_r, g_r, bln_r, o_r):
        ev = e_r[...]
        h = jnp.maximum(gm_r[0] + gm_r[1] + _dot(ev, w1e_r[...]) + b1_r[...], 0.0)
        h = jnp.maximum(_dot(h, w2_r[...]) + b2_r[...], 0.0)
        h = _dot(h, w3_r[...]) + b3_r[...]
        o_r[...] = ev + _ln(h, g_r[...], bln_r[...])

    return pl.pallas_call(
        body,
        grid=(E_PAD // RE,),
        in_specs=[
            pl.BlockSpec((NC, RE, H), lambda i: (0, i, 0)), _rspec(RE, H),
            _wspec((H, H)), _wspec((1, H)), _wspec((H, H)), _wspec((1, H)),
            _wspec((H, H)), _wspec((1, H)), _wspec((1, H)), _wspec((1, H)),
        ],
        out_specs=_rspec(RE, H),
        out_shape=jax.ShapeDtypeStruct((E_PAD, H), _f32),
    )(gm, e, w1e, b1, w2, b2, w3, b3, g, bln)


def _tc_node_block(node, sk, sp, w1n, w1a, b1, w2, b2, w3, b3, g, bln, wsn, wdn):
    have_next = wsn is not None

    def body(n_r, sk_r, sp_r, w1n_r, w1a_r, b1_r, w2_r, b2_r, w3_r, b3_r,
             g_r, bln_r, *rest):
        if have_next:
            wsn_r, wdn_r, o_r, ab_r = rest
        else:
            (o_r,) = rest
        nv = n_r[...]
        agg = (sk_r[0] + sk_r[1]) - (sp_r[0] + sp_r[1])
        h = jnp.maximum(_dot(nv, w1n_r[...]) + _dot(agg, w1a_r[...]) + b1_r[...], 0.0)
        h = jnp.maximum(_dot(h, w2_r[...]) + b2_r[...], 0.0)
        h = _dot(h, w3_r[...]) + b3_r[...]
        nn = nv + _ln(h, g_r[...], bln_r[...])
        o_r[...] = nn
        if have_next:
            ab_r[0] = _dot(nn, wsn_r[...])
            ab_r[1] = _dot(nn, wdn_r[...])

    sspec = pl.BlockSpec((NC, RN, H), lambda i: (0, i, 0))
    in_specs = [
        _rspec(RN, H), sspec, sspec,
        _wspec((H, H)), _wspec((H, H)), _wspec((1, H)), _wspec((H, H)),
        _wspec((1, H)), _wspec((H, H)), _wspec((1, H)), _wspec((1, H)),
        _wspec((1, H)),
    ]
    args = [node, sk, sp, w1n, w1a, b1, w2, b2, w3, b3, g, bln]
    nshape = jax.ShapeDtypeStruct((N, H), _f32)
    if have_next:
        in_specs += [_wspec((H, H)), _wspec((H, H))]
        args += [wsn, wdn]
        out_specs = (_rspec(RN, H), pl.BlockSpec((NC, RN, H), lambda i: (0, i, 0)))
        out_shape = (nshape, jax.ShapeDtypeStruct((NC, N, H), _f32))
    else:
        out_specs = _rspec(RN, H)
        out_shape = nshape
    return pl.pallas_call(
        body,
        grid=(N // RN,),
        in_specs=in_specs,
        out_specs=out_specs,
        out_shape=out_shape,
    )(*args)


def _tc_enc_node(x, ws, bs, g, bln, wsn, wdn):
    def body(x_r, w0, b0, w1, b1, w2, b2, g_r, bln_r, wsn_r, wdn_r,
             o_r, ab_r):
        h = jnp.maximum(_dot(x_r[...], w0[...]) + b0[...], 0.0)
        h = jnp.maximum(_dot(h, w1[...]) + b1[...], 0.0)
        h = _dot(h, w2[...]) + b2[...]
        nn = _ln(h, g_r[...], bln_r[...])
        o_r[...] = nn
        ab_r[0] = _dot(nn, wsn_r[...])
        ab_r[1] = _dot(nn, wdn_r[...])

    nshape = jax.ShapeDtypeStruct((N, H), _f32)
    return pl.pallas_call(
        body,
        grid=(N // RN,),
        in_specs=[_rspec(RN, H)] + [
            _wspec((H, H)), _wspec((1, H)), _wspec((H, H)), _wspec((1, H)),
            _wspec((H, H)), _wspec((1, H)), _wspec((1, H)), _wspec((1, H)),
            _wspec((H, H)), _wspec((H, H)),
        ],
        out_specs=(_rspec(RN, H), pl.BlockSpec((NC, RN, H), lambda i: (0, i, 0))),
        out_shape=(nshape, jax.ShapeDtypeStruct((NC, N, H), _f32)),
    )(x, ws[0][0], bs[0], ws[1][0], bs[1], ws[2][0], bs[2], g, bln, wsn, wdn)


def _tc_enc_edge(ea, ws, bs, g, bln):
    ein = ea.shape[1]

    def body(e_r, w0, b0, w1, b1, w2, b2, g_r, bln_r, o_r):
        h = jnp.maximum(_dot(e_r[...], w0[...]) + b0[...], 0.0)
        h = jnp.maximum(_dot(h, w1[...]) + b1[...], 0.0)
        h = _dot(h, w2[...]) + b2[...]
        o_r[...] = _ln(h, g_r[...], bln_r[...])

    return pl.pallas_call(
        body,
        grid=(E_PAD // RE,),
        in_specs=[_rspec(RE, ein)] + [
            _wspec((ein, H)), _wspec((1, H)), _wspec((H, H)), _wspec((1, H)),
            _wspec((H, H)), _wspec((1, H)), _wspec((1, H)), _wspec((1, H)),
        ],
        out_specs=_rspec(RE, H),
        out_shape=jax.ShapeDtypeStruct((E_PAD, H), _f32),
    )(ea, ws[0][0], bs[0], ws[1][0], bs[1], ws[2][0], bs[2], g, bln)


def _tc_decoder(node, w0, b0, w1p, b1p):
    def body(n_r, w0_r, b0_r, w1_r, b1_r, o_r):
        h = jnp.maximum(_dot(n_r[...], w0_r[...]) + b0_r[...], 0.0)
        o_r[...] = _dot(h, w1_r[...]) + b1_r[...]

    return pl.pallas_call(
        body,
        grid=(N // RN,),
        in_specs=[_rspec(RN, H), _wspec((H, H)), _wspec((1, H)),
                  _wspec((H, H)), _wspec((1, H))],
        out_specs=_rspec(RN, H),
        out_shape=jax.ShapeDtypeStruct((N, H), _f32),
    )(node, w0, b0, w1p, b1p)


def kernel(x, edge_index, edge_attr, params):
    src = edge_index[0].astype(jnp.int32)
    dst = edge_index[1].astype(jnp.int32)
    E = src.shape[0]
    pad = E_PAD - E

    src_p = jnp.concatenate([src, jnp.zeros((pad,), jnp.int32)])
    dst_g = jnp.concatenate([dst, jnp.zeros((pad,), jnp.int32)])
    dst_s = jnp.concatenate([dst, jnp.full((pad,), N, jnp.int32)])
    ea_p = jnp.concatenate(
        [edge_attr, jnp.zeros((pad, edge_attr.shape[1]), edge_attr.dtype)])
    idx_g = jnp.stack([src_p.reshape(NCHUNK, CHUNK),
                       dst_g.reshape(NCHUNK, CHUNK) + N])
    di_s = dst_s.reshape(E_PAD // SCH, SCH)

    r = lambda v: v.reshape(1, H)
    blocks = params["blocks"]

    def eb_split(blk):
        (w1, b1), (w2, b2), (w3, b3) = blk["eb_mlp"]
        g, bln = blk["eb_ln"]
        return (w1[:H], w1[H:2 * H], w1[2 * H:], r(b1), w2, r(b2), w3, r(b3),
                r(g), r(bln))

    enc_n_ws = params["enc_node_mlp"]
    enc_n_bs = [r(b) for (_, b) in enc_n_ws]
    enc_e_ws = params["enc_edge_mlp"]
    enc_e_bs = [r(b) for (_, b) in enc_e_ws]
    g_n, b_n = params["enc_node_ln"]
    g_e, b_e = params["enc_edge_ln"]

    ws1, wd1 = blocks[0]["eb_mlp"][0][0][:H], blocks[0]["eb_mlp"][0][0][H:2 * H]
    node, ab_tab = _tc_enc_node(
        x, enc_n_ws, enc_n_bs, r(g_n), r(b_n), ws1, wd1)
    edge = _tc_enc_edge(ea_p, enc_e_ws, enc_e_bs, r(g_e), r(b_e))

    s_prev = _sc_scatter(edge, di_s)

    for k, blk in enumerate(blocks):
        w1s, w1d, w1e, b1, w2, b2, w3, b3, ge, bline = eb_split(blk)
        gm = _sc_gather(ab_tab.reshape(2 * N, H), idx_g)
        edge = _tc_edge_block(gm, edge, w1e, b1, w2, b2, w3, b3, ge, bline)
        s_k = _sc_scatter(edge, di_s)

        (nw1, nb1), (nw2, nb2), (nw3, nb3) = blk["nb_mlp"]
        gn, blnn = blk["nb_ln"]
        if k + 1 < len(blocks):
            nws, nwd = (blocks[k + 1]["eb_mlp"][0][0][:H],
                        blocks[k + 1]["eb_mlp"][0][0][H:2 * H])
            node, ab_tab = _tc_node_block(
                node, s_k, s_prev, nw1[:H], nw1[H:], r(nb1), nw2, r(nb2),
                nw3, r(nb3), r(gn), r(blnn), nws, nwd)
        else:
            node = _tc_node_block(
                node, s_k, s_prev, nw1[:H], nw1[H:], r(nb1), nw2, r(nb2),
                nw3, r(nb3), r(gn), r(blnn), None, None)
        s_prev = s_k

    (dw0, db0), (dw1, db1) = params["dec_mlp"]
    out_dim = dw1.shape[1]
    w1p = jnp.zeros((H, H), _f32).at[:, :out_dim].set(dw1)
    b1p = jnp.zeros((1, H), _f32).at[0, :out_dim].set(db1)
    out = _tc_decoder(node, dw0, r(db0), w1p, b1p)
    return out[:, :out_dim]
